# baseline (speedup 1.0000x reference)
import functools

import jax
import jax.numpy as jnp
from jax import lax
from jax.experimental import pallas as pl
from jax.experimental.pallas import tpu as pltpu
from jax.experimental.pallas import tpu_sc as plsc

DIM = 64
BATCH = 16384
NC = 2
NS = 16
NW = NC * NS
CHUNK = BATCH // NW
S = 64
NSUB = CHUNK // S
NG = S // 16


def _sqrt16(v):
    x = jnp.maximum(v, jnp.float32(1e-30))
    i = plsc.bitcast(x, jnp.int32)
    y = plsc.bitcast(jnp.int32(0x5F3759DF) - (i >> 1), jnp.float32)
    for _ in range(3):
        y = y * (jnp.float32(1.5) - jnp.float32(0.5) * x * y * y)
    return x * y


def _make_kernel():
    mesh = plsc.VectorSubcoreMesh(core_axis_name="c", subcore_axis_name="s")
    fvmem = lambda shape: pltpu.VMEM(shape, jnp.float32)
    ivmem = lambda shape: pltpu.VMEM(shape, jnp.int32)

    scratch = (
        [ivmem((S,)) for _ in range(12)]
        + [fvmem((S, 2 * DIM)) for _ in range(8)]
        + [fvmem((S,))]
        + [pltpu.SemaphoreType.DMA, pltpu.SemaphoreType.DMA]
    )

    @functools.partial(
        pl.kernel,
        out_type=jax.ShapeDtypeStruct((BATCH,), jnp.float32),
        mesh=mesh,
        scratch_types=scratch,
        compiler_params=pltpu.CompilerParams(
            needs_layout_passes=False, use_tc_tiling_on_sc=True),
    )
    def trans_h(h_hbm, r_hbm, t_hbm, ent_hbm, rel_hbm, nrm_hbm, out_hbm,
                ih0, it0, ir0, jh0, jt0, jr0,
                ih1, it1, ir1, jh1, jt1, jr1,
                hb0, tb0, rb0, nb0, hb1, tb1, rb1, nb1,
                score_buf, sem0, sem1):
        wid = lax.axis_index("s") * NC + lax.axis_index("c")
        base = wid * CHUNK

        idx = ((ih0, it0, ir0, jh0, jt0, jr0),
               (ih1, it1, ir1, jh1, jt1, jr1))
        rows = ((hb0, tb0, rb0, nb0), (hb1, tb1, rb1, nb1))
        sems = (sem0, sem1)

        def fire(setid, sub):
            off = base + sub * S
            ih, it, ir, jh, jt, jr = idx[setid]
            hb, tb, rb, nb = rows[setid]
            pltpu.sync_copy(h_hbm.at[pl.ds(off, S)], ih)
            pltpu.sync_copy(t_hbm.at[pl.ds(off, S)], it)
            pltpu.sync_copy(r_hbm.at[pl.ds(off, S)], ir)
            for g in range(NG):
                sl = pl.ds(g * 16, 16)
                jh[sl] = ih[sl] >> 1
                jt[sl] = it[sl] >> 1
                jr[sl] = ir[sl] >> 1
            sem = sems[setid]
            return [
                pltpu.async_copy(ent_hbm.at[jh], hb, sem),
                pltpu.async_copy(ent_hbm.at[jt], tb, sem),
                pltpu.async_copy(rel_hbm.at[jr], rb, sem),
                pltpu.async_copy(nrm_hbm.at[jr], nb, sem),
            ]

        def compute(setid, sub):
            off = base + sub * S
            ih, it, ir, jh, jt, jr = idx[setid]
            hb, tb, rb, nb = rows[setid]

            def group(g, _):
                rix = g * 16 + lax.iota(jnp.int32, 16)
                sl = pl.ds(g * 16, 16)
                oh = (ih[sl] & 1) * DIM
                ot = (it[sl] & 1) * DIM
                orr = (ir[sl] & 1) * DIM
                zero = jnp.zeros((16,), jnp.float32)

                def dim_body(d, acc):
                    ww, wn, rn, nn = acc
                    hv = plsc.load_gather(hb, [rix, oh + d])
                    tv = plsc.load_gather(tb, [rix, ot + d])
                    rv = plsc.load_gather(rb, [rix, orr + d])
                    nv = plsc.load_gather(nb, [rix, orr + d])
                    w = hv - tv + rv
                    return (ww + w * w, wn + w * nv, rn + rv * nv,
                            nn + nv * nv)

                ww, wn, rn, nn = lax.fori_loop(
                    0, DIM, dim_body, (zero, zero, zero, zero), unroll=8)
                c = (wn - rn) / jnp.maximum(nn, jnp.float32(1e-24))
                s2 = jnp.maximum(ww - 2.0 * c * wn + c * c * nn,
                                 jnp.float32(0.0))
                score_buf[pl.ds(g * 16, 16)] = _sqrt16(s2)
                return 0

            lax.fori_loop(0, NG, group, 0)
            pltpu.sync_copy(score_buf, out_hbm.at[pl.ds(off, S)])

        cps = fire(0, 0)
        for sub in range(NSUB):
            cur = sub & 1
            nxt_cps = fire(1 - cur, sub + 1) if sub + 1 < NSUB else None
            for c in cps:
                c.wait()
            compute(cur, sub)
            cps = nxt_cps

    return trans_h


_trans_h = _make_kernel()


@jax.jit
def kernel(h, r, t, entity_emb, relation_emb, relation_norm):
    n_ent, dim = entity_emb.shape
    n_rel = relation_emb.shape[0]
    ent2 = entity_emb.reshape(n_ent // 2, 2 * dim)
    rel2 = relation_emb.reshape(n_rel // 2, 2 * dim)
    nrm2 = relation_norm.reshape(n_rel // 2, 2 * dim)
    return _trans_h(h.astype(jnp.int32), r.astype(jnp.int32),
                    t.astype(jnp.int32), ent2, rel2, nrm2)


# speedup vs baseline: 1.4793x; 1.4793x over previous
import functools

import jax
import jax.numpy as jnp
from jax import lax
from jax.experimental import pallas as pl
from jax.experimental.pallas import tpu as pltpu
from jax.experimental.pallas import tpu_sc as plsc

DIM = 64
BATCH = 16384
NC = 2
NS = 16
NW = NC * NS
CHUNK = BATCH // NW
S = 16
NSUB = CHUNK // S


def _sqrt16(v):
    x = jnp.maximum(v, jnp.float32(1e-30))
    i = plsc.bitcast(x, jnp.int32)
    y = plsc.bitcast(jnp.int32(0x5F3759DF) - (i >> 1), jnp.float32)
    for _ in range(3):
        y = y * (jnp.float32(1.5) - jnp.float32(0.5) * x * y * y)
    return x * y


def _make_kernel():
    mesh = plsc.VectorSubcoreMesh(core_axis_name="c", subcore_axis_name="s")
    fvmem = lambda shape: pltpu.VMEM(shape, jnp.float32)
    ivmem = lambda shape: pltpu.VMEM(shape, jnp.int32)

    scratch = (
        [ivmem((S,)) for _ in range(8)]
        + [fvmem((S, 8, DIM)) for _ in range(4)]
        + [fvmem((S, 2 * DIM)) for _ in range(4)]
        + [fvmem((S,))]
        + [pltpu.SemaphoreType.DMA, pltpu.SemaphoreType.DMA]
    )

    @functools.partial(
        pl.kernel,
        out_type=jax.ShapeDtypeStruct((BATCH,), jnp.float32),
        mesh=mesh,
        scratch_types=scratch,
        compiler_params=pltpu.CompilerParams(
            needs_layout_passes=False, use_tc_tiling_on_sc=True),
    )
    def trans_h(h_hbm, r_hbm, t_hbm, ent_hbm, rel_hbm, nrm_hbm, out_hbm,
                ih0, it0, ir0, jr0, ih1, it1, ir1, jr1,
                hb0, tb0, hb1, tb1, rb0, nb0, rb1, nb1,
                score_buf, sem0, sem1):
        wid = lax.axis_index("s") * NC + lax.axis_index("c")
        base = wid * CHUNK

        idx = ((ih0, it0, ir0, jr0), (ih1, it1, ir1, jr1))
        slabs = ((hb0, tb0), (hb1, tb1))
        rel_rows = ((rb0, nb0), (rb1, nb1))
        sems = (sem0, sem1)

        def fire(setid, sub):
            off = base + sub * S
            ih, it, ir, jr = idx[setid]
            hb, tb = slabs[setid]
            rb, nb = rel_rows[setid]
            sem = sems[setid]
            pltpu.sync_copy(h_hbm.at[pl.ds(off, S)], ih)
            pltpu.sync_copy(t_hbm.at[pl.ds(off, S)], it)
            pltpu.sync_copy(r_hbm.at[pl.ds(off, S)], ir)
            jr[...] = ir[...] >> 1
            vh = (ih[...] >> 3) * 8
            vt = (it[...] >> 3) * 8
            for k in range(16):
                eh = pl.multiple_of(vh[k], 8)
                et = pl.multiple_of(vt[k], 8)
                pltpu.async_copy(ent_hbm.at[pl.ds(eh, 8)], hb.at[k], sem)
                pltpu.async_copy(ent_hbm.at[pl.ds(et, 8)], tb.at[k], sem)
            pltpu.async_copy(rel_hbm.at[jr], rb, sem)
            pltpu.async_copy(nrm_hbm.at[jr], nb, sem)

        def wait_all(setid):
            hb, tb = slabs[setid]
            rb, nb = rel_rows[setid]
            sem = sems[setid]
            for k in range(16):
                pltpu.make_async_copy(
                    ent_hbm.at[pl.ds(0, 8)], hb.at[k], sem).wait()
                pltpu.make_async_copy(
                    ent_hbm.at[pl.ds(0, 8)], tb.at[k], sem).wait()
            pltpu.make_async_copy(rel_hbm.at[pl.ds(0, S)], rb, sem).wait()
            pltpu.make_async_copy(nrm_hbm.at[pl.ds(0, S)], nb, sem).wait()

        def compute(setid, sub):
            off = base + sub * S
            ih, it, ir, jr = idx[setid]
            hb, tb = slabs[setid]
            rb, nb = rel_rows[setid]

            rix = lax.iota(jnp.int32, 16)
            rowh = ih[...] & 7
            rowt = it[...] & 7
            orr = (ir[...] & 1) * DIM
            zero = jnp.zeros((16,), jnp.float32)

            def dim_body(d, acc):
                ww, wn, rn, nn = acc
                col = jnp.zeros((16,), jnp.int32) + d
                hv = plsc.load_gather(hb, [rix, rowh, col])
                tv = plsc.load_gather(tb, [rix, rowt, col])
                rv = plsc.load_gather(rb, [rix, orr + d])
                nv = plsc.load_gather(nb, [rix, orr + d])
                w = hv - tv + rv
                return (ww + w * w, wn + w * nv, rn + rv * nv,
                        nn + nv * nv)

            ww, wn, rn, nn = lax.fori_loop(
                0, DIM, dim_body, (zero, zero, zero, zero), unroll=8)
            c = (wn - rn) / jnp.maximum(nn, jnp.float32(1e-24))
            s2 = jnp.maximum(ww - 2.0 * c * wn + c * c * nn,
                             jnp.float32(0.0))
            score_buf[...] = _sqrt16(s2)
            pltpu.sync_copy(score_buf, out_hbm.at[pl.ds(off, S)])

        fire(0, 0)

        def body(p, _):
            sub0 = 2 * p
            sub1 = 2 * p + 1
            fire(1, sub1)
            wait_all(0)
            compute(0, sub0)
            fire(0, jnp.minimum(sub1 + 1, NSUB - 1))
            wait_all(1)
            compute(1, sub1)
            return 0

        lax.fori_loop(0, NSUB // 2, body, 0)
        wait_all(0)

    return trans_h


_trans_h = _make_kernel()


@jax.jit
def kernel(h, r, t, entity_emb, relation_emb, relation_norm):
    n_rel = relation_emb.shape[0]
    rel2 = relation_emb.reshape(n_rel // 2, 2 * DIM)
    nrm2 = relation_norm.reshape(n_rel // 2, 2 * DIM)
    return _trans_h(h.astype(jnp.int32), r.astype(jnp.int32),
                    t.astype(jnp.int32), entity_emb, rel2, nrm2)


# speedup vs baseline: 1.6065x; 1.0860x over previous
import functools

import jax
import jax.numpy as jnp
from jax import lax
from jax.experimental import pallas as pl
from jax.experimental.pallas import tpu as pltpu
from jax.experimental.pallas import tpu_sc as plsc

DIM = 64
BATCH = 16384
NC = 2
NS = 16
NW = NC * NS
CHUNK = BATCH // NW
S = 16
NSUB = CHUNK // S


def _sqrt16(v):
    x = jnp.maximum(v, jnp.float32(1e-30))
    i = plsc.bitcast(x, jnp.int32)
    y = plsc.bitcast(jnp.int32(0x5F3759DF) - (i >> 1), jnp.float32)
    for _ in range(3):
        y = y * (jnp.float32(1.5) - jnp.float32(0.5) * x * y * y)
    return x * y


def _make_kernel():
    mesh = plsc.VectorSubcoreMesh(core_axis_name="c", subcore_axis_name="s")
    fvmem = lambda shape: pltpu.VMEM(shape, jnp.float32)
    ivmem = lambda shape: pltpu.VMEM(shape, jnp.int32)

    scratch = (
        [ivmem((CHUNK,)) for _ in range(4)]
        + [fvmem((8 * S, DIM)) for _ in range(4)]
        + [fvmem((S, 2 * DIM)) for _ in range(4)]
        + [fvmem((CHUNK,))]
        + [pltpu.SemaphoreType.DMA, pltpu.SemaphoreType.DMA]
    )

    @functools.partial(
        pl.kernel,
        out_type=jax.ShapeDtypeStruct((BATCH,), jnp.float32),
        mesh=mesh,
        scratch_types=scratch,
        compiler_params=pltpu.CompilerParams(
            needs_layout_passes=False, use_tc_tiling_on_sc=True),
    )
    def trans_h(h_hbm, r_hbm, t_hbm, ent_hbm, rel_hbm, nrm_hbm, out_hbm,
                ihx, itx, irx, jrx,
                hb0, tb0, hb1, tb1, rb0, nb0, rb1, nb1,
                score_all, sem0, sem1):
        wid = lax.axis_index("s") * NC + lax.axis_index("c")
        base = wid * CHUNK

        slabs = ((hb0, tb0), (hb1, tb1))
        rel_rows = ((rb0, nb0), (rb1, nb1))
        sems = (sem0, sem1)

        pltpu.sync_copy(h_hbm.at[pl.ds(base, CHUNK)], ihx)
        pltpu.sync_copy(t_hbm.at[pl.ds(base, CHUNK)], itx)
        pltpu.sync_copy(r_hbm.at[pl.ds(base, CHUNK)], irx)

        def halve(g, _):
            sl = pl.ds(g * 16, 16)
            jrx[sl] = irx[sl] >> 1
            return 0

        lax.fori_loop(0, CHUNK // 16, halve, 0)

        def fire(setid, sub):
            hb, tb = slabs[setid]
            rb, nb = rel_rows[setid]
            sem = sems[setid]
            sl = pl.ds(sub * S, 16)
            vh = (ihx[sl] >> 3) * 8
            vt = (itx[sl] >> 3) * 8
            for k in range(16):
                eh = pl.multiple_of(vh[k], 8)
                et = pl.multiple_of(vt[k], 8)
                pltpu.async_copy(ent_hbm.at[pl.ds(eh, 8)],
                                 hb.at[pl.ds(k * 8, 8)], sem)
                pltpu.async_copy(ent_hbm.at[pl.ds(et, 8)],
                                 tb.at[pl.ds(k * 8, 8)], sem)
            jr_ref = jrx.at[pl.ds(sub * S, S)]
            pltpu.async_copy(rel_hbm.at[jr_ref], rb, sem)
            pltpu.async_copy(nrm_hbm.at[jr_ref], nb, sem)

        def wait_all(setid):
            hb, tb = slabs[setid]
            rb, nb = rel_rows[setid]
            sem = sems[setid]
            pltpu.make_async_copy(ent_hbm.at[pl.ds(0, 8 * S)], hb, sem).wait()
            pltpu.make_async_copy(ent_hbm.at[pl.ds(0, 8 * S)], tb, sem).wait()
            pltpu.make_async_copy(rel_hbm.at[pl.ds(0, S)], rb, sem).wait()
            pltpu.make_async_copy(nrm_hbm.at[pl.ds(0, S)], nb, sem).wait()

        def compute(setid, sub):
            hb, tb = slabs[setid]
            rb, nb = rel_rows[setid]
            sl = pl.ds(sub * S, 16)

            rix8 = lax.iota(jnp.int32, 16) * 8
            rix = lax.iota(jnp.int32, 16)
            rowh = rix8 + (ihx[sl] & 7)
            rowt = rix8 + (itx[sl] & 7)
            orr = (irx[sl] & 1) * DIM
            zero = jnp.zeros((16,), jnp.float32)

            def dim_body(d, acc):
                ww, wn, rn, nn = acc
                col = jnp.zeros((16,), jnp.int32) + d
                hv = plsc.load_gather(hb, [rowh, col])
                tv = plsc.load_gather(tb, [rowt, col])
                rv = plsc.load_gather(rb, [rix, orr + d])
                nv = plsc.load_gather(nb, [rix, orr + d])
                w = hv - tv + rv
                return (ww + w * w, wn + w * nv, rn + rv * nv,
                        nn + nv * nv)

            ww, wn, rn, nn = lax.fori_loop(
                0, DIM, dim_body, (zero, zero, zero, zero), unroll=8)
            c = (wn - rn) / jnp.maximum(nn, jnp.float32(1e-24))
            s2 = jnp.maximum(ww - 2.0 * c * wn + c * c * nn,
                             jnp.float32(0.0))
            score_all[sl] = _sqrt16(s2)

        fire(0, 0)

        def body(p, _):
            sub0 = 2 * p
            sub1 = 2 * p + 1
            fire(1, sub1)
            wait_all(0)
            compute(0, sub0)
            fire(0, jnp.minimum(sub1 + 1, NSUB - 1))
            wait_all(1)
            compute(1, sub1)
            return 0

        lax.fori_loop(0, NSUB // 2, body, 0)
        wait_all(0)
        pltpu.sync_copy(score_all, out_hbm.at[pl.ds(base, CHUNK)])

    return trans_h


_trans_h = _make_kernel()


@jax.jit
def kernel(h, r, t, entity_emb, relation_emb, relation_norm):
    n_rel = relation_emb.shape[0]
    rel2 = relation_emb.reshape(n_rel // 2, 2 * DIM)
    nrm2 = relation_norm.reshape(n_rel // 2, 2 * DIM)
    return _trans_h(h.astype(jnp.int32), r.astype(jnp.int32),
                    t.astype(jnp.int32), entity_emb, rel2, nrm2)
